# baseline (speedup 1.0000x reference)
import jax
import jax.numpy as jnp
from jax.experimental import pallas as pl
from jax.experimental.pallas import tpu as pltpu

_B, _C, _H, _W = 8, 384, 200, 176
_O_PAD = 32
_HB = 40


def _head_kernel(x_ref, w_ref, b_ref, cls_ref, reg_ref, dir_ref):
    xb = x_ref[0].reshape(_C, _HB * _W)
    acc = jax.lax.dot_general(
        w_ref[...], xb,
        dimension_numbers=(((1,), (0,)), ((), ())),
        preferred_element_type=jnp.float32,
    )
    acc = acc + b_ref[...]
    cls_ref[0] = acc[0:2].reshape(2, _HB, _W)
    reg_ref[0] = acc[2:16].reshape(14, _HB, _W)
    dir_ref[0] = acc[16:20].reshape(4, _HB, _W)


def kernel(x, W_cls, b_cls, W_reg, b_reg, W_dir, b_dir):
    w = jnp.concatenate([W_cls, W_reg, W_dir], axis=1).T
    w = jnp.pad(w, ((0, _O_PAD - w.shape[0]), (0, 0)))
    b = jnp.concatenate([b_cls, b_reg, b_dir])
    b = jnp.pad(b, (0, _O_PAD - b.shape[0]))[:, None]

    n_blocks = _H // _HB

    cls_o, reg_o, dir_o = pl.pallas_call(
        _head_kernel,
        grid=(_B, n_blocks),
        in_specs=[
            pl.BlockSpec((1, _C, _HB, _W), lambda bi, hi: (bi, 0, hi, 0)),
            pl.BlockSpec((_O_PAD, _C), lambda bi, hi: (0, 0)),
            pl.BlockSpec((_O_PAD, 1), lambda bi, hi: (0, 0)),
        ],
        out_specs=[
            pl.BlockSpec((1, 2, _HB, _W), lambda bi, hi: (bi, 0, hi, 0)),
            pl.BlockSpec((1, 14, _HB, _W), lambda bi, hi: (bi, 0, hi, 0)),
            pl.BlockSpec((1, 4, _HB, _W), lambda bi, hi: (bi, 0, hi, 0)),
        ],
        out_shape=[
            jax.ShapeDtypeStruct((_B, 2, _H, _W), jnp.float32),
            jax.ShapeDtypeStruct((_B, 14, _H, _W), jnp.float32),
            jax.ShapeDtypeStruct((_B, 4, _H, _W), jnp.float32),
        ],
        compiler_params=pltpu.CompilerParams(
            dimension_semantics=("parallel", "parallel"),
        ),
    )(x, w, b)

    return (cls_o, reg_o, dir_o)
